# baseline (speedup 1.0000x reference)
import functools

import jax
import jax.numpy as jnp
from jax import lax
from jax.experimental import pallas as pl
from jax.experimental.pallas import tpu as pltpu
from jax.experimental.pallas import tpu_sc as plsc

B = 1024
D = 16
NC = 2
NS = 16
NW = NC * NS
S = 512
RPW = S // NW
NCH = B // D
TR = 256


def _sigmoid(v):
    return 1.0 / (1.0 + jnp.exp(-v))


def _tree_add(terms):
    terms = list(terms)
    while len(terms) > 1:
        terms = [a + b for a, b in zip(terms[::2], terms[1::2])] + (
            [terms[-1]] if len(terms) % 2 else []
        )
    return terms[0]


def _sc_half(x1, x2t):
    mesh = plsc.VectorSubcoreMesh(
        core_axis_name="c", subcore_axis_name="s", num_cores=NC, num_subcores=NS
    )

    @functools.partial(
        pl.kernel,
        out_type=(
            jax.ShapeDtypeStruct((S, B), jnp.float32),
            jax.ShapeDtypeStruct((B, S), jnp.float32),
        ),
        mesh=mesh,
        compiler_params=pltpu.CompilerParams(
            use_tc_tiling_on_sc=False, needs_layout_passes=False
        ),
        scratch_types=[
            pltpu.VMEM((RPW, D), jnp.float32),
            pltpu.VMEM((D, B), jnp.float32),
            pltpu.VMEM((B,), jnp.float32),
            pltpu.VMEM((RPW, B), jnp.float32),
            pltpu.VMEM((B, RPW), jnp.float32),
            pltpu.SMEM((RPW, D + 1), jnp.float32),
        ],
    )
    def sc_kernel(x1_hbm, x2t_hbm, out1_hbm, out2_hbm,
                  rows_v, bt_v, sb_v, out_v, outt_v, scal_s):
        wid = lax.axis_index("s") * NC + lax.axis_index("c")
        base = wid * RPW

        pltpu.sync_copy(x1_hbm.at[pl.ds(base, RPW)], rows_v)
        pltpu.sync_copy(x2t_hbm, bt_v)

        def sb_body(c, carry):
            off = c * D
            vals = [_sigmoid(bt_v[d, pl.ds(off, D)]) for d in range(D)]
            for d in range(D):
                bt_v[d, pl.ds(off, D)] = vals[d]
            sb_v[pl.ds(off, D)] = _tree_add(vals)
            return carry

        lax.fori_loop(0, NCH, sb_body, 0, unroll=False)

        def arow_body(i, carry):
            v = _sigmoid(rows_v[i, :])
            sa = v[0]
            scal_s[i, 0] = v[0]
            for d in range(1, D):
                scal_s[i, d] = v[d]
                sa = sa + v[d]
            scal_s[i, D] = sa
            return carry

        lax.fori_loop(0, RPW, arow_body, 0, unroll=False)

        def col_body(c, carry):
            off = c * D
            bts = [bt_v[d, pl.ds(off, D)] for d in range(D)]
            sbc = sb_v[pl.ds(off, D)]
            ridx = lax.iota(jnp.int32, D) + off
            zsplat = jnp.full((D,), 0, jnp.int32)

            def row_body(i, icarry):
                mins = _tree_add(
                    jnp.minimum(scal_s[i, d], bts[d]) for d in range(D)
                )
                sim = mins / (scal_s[i, D] + sbc - mins)
                out_v[i, pl.ds(off, D)] = sim
                plsc.store_scatter(outt_v, [ridx, zsplat + i], sim)
                return icarry

            lax.fori_loop(0, RPW, row_body, 0, unroll=False)
            return carry

        lax.fori_loop(0, NCH, col_body, 0, unroll=False)

        pltpu.sync_copy(out_v, out1_hbm.at[pl.ds(base, RPW)])
        pltpu.sync_copy(outt_v, out2_hbm.at[:, pl.ds(base, RPW)])

    return sc_kernel(x1, x2t)


def _tc_half(x1, x2):
    def tc_kernel(x1_ref, x2_ref, o1_ref, o2_ref):
        a = _sigmoid(x1_ref[...])
        b = _sigmoid(x2_ref[...])
        sa = jnp.sum(a, axis=1)
        sb = jnp.sum(b, axis=1)
        mins = jnp.zeros((TR, B), jnp.float32)
        for d in range(D):
            mins = mins + jnp.minimum(a[:, d:d + 1], b[:, d][None, :])
        sim = mins / (sa[:, None] + sb[None, :] - mins)
        o1_ref[...] = sim
        o2_ref[...] = sim.T

    nblk = (B - S) // TR
    return pl.pallas_call(
        tc_kernel,
        grid=(nblk,),
        in_specs=[
            pl.BlockSpec((TR, D), lambda g: (g + S // TR, 0)),
            pl.BlockSpec((B, D), lambda g: (0, 0)),
        ],
        out_specs=[
            pl.BlockSpec((TR, B), lambda g: (g, 0)),
            pl.BlockSpec((B, TR), lambda g: (0, g)),
        ],
        out_shape=[
            jax.ShapeDtypeStruct((B - S, B), jnp.float32),
            jax.ShapeDtypeStruct((B, B - S), jnp.float32),
        ],
    )(x1, x2)


def kernel(x1, x2):
    x2t = x2.T
    sc1, sc2 = _sc_half(x1, x2t)
    tc1, tc2 = _tc_half(x1, x2)
    out1 = jnp.concatenate([sc1, tc1], axis=0)
    out2 = jnp.concatenate([sc2, tc2], axis=1)
    return (out1, out2)


# speedup vs baseline: 1.1232x; 1.1232x over previous
import functools

import jax
import jax.numpy as jnp
from jax import lax
from jax.experimental import pallas as pl
from jax.experimental.pallas import tpu as pltpu
from jax.experimental.pallas import tpu_sc as plsc

B = 1024
D = 16
NC = 2
NS = 16
NW = NC * NS
S = 256
RPW = S // NW
NCH = B // D
TR = 256


def _sigmoid(v):
    return 1.0 / (1.0 + jnp.exp(-v))


def _tree_add(terms):
    terms = list(terms)
    while len(terms) > 1:
        terms = [a + b for a, b in zip(terms[::2], terms[1::2])] + (
            [terms[-1]] if len(terms) % 2 else []
        )
    return terms[0]


def _sc_half(x1, x2t):
    mesh = plsc.VectorSubcoreMesh(
        core_axis_name="c", subcore_axis_name="s", num_cores=NC, num_subcores=NS
    )

    @functools.partial(
        pl.kernel,
        out_type=(
            jax.ShapeDtypeStruct((S, B), jnp.float32),
            jax.ShapeDtypeStruct((B, S), jnp.float32),
        ),
        mesh=mesh,
        compiler_params=pltpu.CompilerParams(
            use_tc_tiling_on_sc=False, needs_layout_passes=False
        ),
        scratch_types=[
            pltpu.VMEM((RPW, D), jnp.float32),
            pltpu.VMEM((D, B), jnp.float32),
            pltpu.VMEM((B,), jnp.float32),
            pltpu.VMEM((RPW, B), jnp.float32),
            pltpu.VMEM((B, RPW), jnp.float32),
            pltpu.SMEM((RPW, D + 1), jnp.float32),
        ],
    )
    def sc_kernel(x1_hbm, x2t_hbm, out1_hbm, out2_hbm,
                  rows_v, bt_v, sb_v, out_v, outt_v, scal_s):
        wid = lax.axis_index("s") * NC + lax.axis_index("c")
        base = wid * RPW

        pltpu.sync_copy(x1_hbm.at[pl.ds(base, RPW)], rows_v)
        pltpu.sync_copy(x2t_hbm, bt_v)

        def sb_body(c, carry):
            off = c * D
            vals = [_sigmoid(bt_v[d, pl.ds(off, D)]) for d in range(D)]
            for d in range(D):
                bt_v[d, pl.ds(off, D)] = vals[d]
            sb_v[pl.ds(off, D)] = _tree_add(vals)
            return carry

        lax.fori_loop(0, NCH, sb_body, 0, unroll=False)

        def arow_body(i, carry):
            v = _sigmoid(rows_v[i, :])
            sa = v[0]
            scal_s[i, 0] = v[0]
            for d in range(1, D):
                scal_s[i, d] = v[d]
                sa = sa + v[d]
            scal_s[i, D] = sa
            return carry

        lax.fori_loop(0, RPW, arow_body, 0, unroll=False)

        def col_body(c, carry):
            off = c * D
            bts = [bt_v[d, pl.ds(off, D)] for d in range(D)]
            sbc = sb_v[pl.ds(off, D)]
            ridx = lax.iota(jnp.int32, D) + off
            zsplat = jnp.full((D,), 0, jnp.int32)

            def row_body(i, icarry):
                mins = _tree_add(
                    jnp.minimum(scal_s[i, d], bts[d]) for d in range(D)
                )
                sim = mins / (scal_s[i, D] + sbc - mins)
                out_v[i, pl.ds(off, D)] = sim
                plsc.store_scatter(outt_v, [ridx, zsplat + i], sim)
                return icarry

            lax.fori_loop(0, RPW, row_body, 0, unroll=False)
            return carry

        lax.fori_loop(0, NCH, col_body, 0, unroll=False)

        pltpu.sync_copy(out_v, out1_hbm.at[pl.ds(base, RPW)])
        pltpu.sync_copy(outt_v, out2_hbm.at[:, pl.ds(base, RPW)])

    return sc_kernel(x1, x2t)


def _tc_half(x1, x2):
    def tc_kernel(x1_ref, x2_ref, o1_ref, o2_ref):
        a = _sigmoid(x1_ref[...])
        b = _sigmoid(x2_ref[...])
        sa = jnp.sum(a, axis=1)
        sb = jnp.sum(b, axis=1)
        mins = jnp.zeros((TR, B), jnp.float32)
        for d in range(D):
            mins = mins + jnp.minimum(a[:, d:d + 1], b[:, d][None, :])
        sim = mins / (sa[:, None] + sb[None, :] - mins)
        o1_ref[...] = sim
        o2_ref[...] = sim.T

    nblk = (B - S) // TR
    return pl.pallas_call(
        tc_kernel,
        grid=(nblk,),
        in_specs=[
            pl.BlockSpec((TR, D), lambda g: (g + S // TR, 0)),
            pl.BlockSpec((B, D), lambda g: (0, 0)),
        ],
        out_specs=[
            pl.BlockSpec((TR, B), lambda g: (g, 0)),
            pl.BlockSpec((B, TR), lambda g: (0, g)),
        ],
        out_shape=[
            jax.ShapeDtypeStruct((B - S, B), jnp.float32),
            jax.ShapeDtypeStruct((B, B - S), jnp.float32),
        ],
    )(x1, x2)


def kernel(x1, x2):
    x2t = x2.T
    sc1, sc2 = _sc_half(x1, x2t)
    tc1, tc2 = _tc_half(x1, x2)
    out1 = jnp.concatenate([sc1, tc1], axis=0)
    out2 = jnp.concatenate([sc2, tc2], axis=1)
    return (out1, out2)
